# baseline (speedup 1.0000x reference)
import functools

import jax
import jax.numpy as jnp
from jax import lax
from jax.experimental import pallas as pl
from jax.experimental.pallas import tpu as pltpu
from jax.experimental.pallas import tpu_sc as plsc

L = 16
NC, NS = 2, 16
NW = NC * NS
K = 16
CELL = 1.0
NX = 20
NY = 20
NBATCH = 4
NB = NBATCH * NX * NY
NBP = 1664
CH = 4096


def _make_body(n_ref, n_query):
  qw = n_query // NW

  def body(rb_h, rx_h, ry_h, rz_h, qb_h, qx_h, qy_h, qz_h, rad_h, nn_h,
           out_ri, out_qi,
           crb, crx, cry, crz, sx, sy, sz, sidx,
           counts, starts, cursors,
           qbv, qxv, qyv, qzv, cxlo_a, cxhi_a, base_a, dy1_a,
           stage_ri, stage_qi, cur_d, cur_i, tmpa, tmpb, parv, nnv):
    wid = lax.axis_index("c") * NS + lax.axis_index("s")
    iota = lax.iota(jnp.int32, L)
    zeros16 = jnp.zeros((L,), jnp.int32)

    pltpu.sync_copy(rad_h, parv)
    pltpu.sync_copy(nn_h, nnv)
    rv = parv[...]
    r2v = rv * rv

    def zero_body(k, _):
      plsc.store_scatter(counts, [k * L + iota], zeros16)
      return 0

    lax.fori_loop(0, NBP // L, zero_body, 0)

    scope = jax.named_scope

    def bucket_of(bb, xx, yy):
      cx = jnp.clip((xx * (1.0 / CELL)).astype(jnp.int32), 0, NX - 1)
      cy = jnp.clip((yy * (1.0 / CELL)).astype(jnp.int32), 0, NY - 1)
      return (bb.astype(jnp.int32) * NX + cx) * NY + cy

    for c in range(0, n_ref, CH):
     with scope("p1a_count"):
      pltpu.sync_copy(rb_h.at[pl.ds(c, CH)], crb)
      pltpu.sync_copy(rx_h.at[pl.ds(c, CH)], crx)
      pltpu.sync_copy(ry_h.at[pl.ds(c, CH)], cry)

      def cnt_body(j, _):
        idxv = j * L + iota
        bb = plsc.load_gather(crb, [idxv])
        xx = plsc.load_gather(crx, [idxv])
        yy = plsc.load_gather(cry, [idxv])
        bkt = bucket_of(bb, xx, yy)
        rank, is_last = plsc.scan_count(bkt)
        plsc.addupdate_scatter(counts, [bkt], rank, mask=is_last)
        return 0

      lax.fori_loop(0, CH // L, cnt_body, 0, unroll=4)

    def psum_body(k, carry):
      idxv = k * L + iota
      cnt = plsc.load_gather(counts, [idxv])
      cm = plsc.cumsum(cnt)
      plsc.store_scatter(starts, [idxv], carry + cm - cnt)
      tmpa[...] = cm
      return carry + plsc.load_gather(tmpa, [jnp.full((L,), L - 1, jnp.int32)])

    lax.fori_loop(0, NBP // L, psum_body, zeros16)

    def ccopy_body(k, _):
      idxv = k * L + iota
      plsc.store_scatter(cursors, [idxv], plsc.load_gather(starts, [idxv]))
      return 0

    lax.fori_loop(0, NBP // L, ccopy_body, 0)

    for c in range(0, n_ref, CH):
     with scope("p1c_scatter"):
      pltpu.sync_copy(rb_h.at[pl.ds(c, CH)], crb)
      pltpu.sync_copy(rx_h.at[pl.ds(c, CH)], crx)
      pltpu.sync_copy(ry_h.at[pl.ds(c, CH)], cry)
      pltpu.sync_copy(rz_h.at[pl.ds(c, CH)], crz)

      def sc_body(j, _):
        idxv = j * L + iota
        bb = plsc.load_gather(crb, [idxv])
        xx = plsc.load_gather(crx, [idxv])
        yy = plsc.load_gather(cry, [idxv])
        zz = plsc.load_gather(crz, [idxv])
        bkt = bucket_of(bb, xx, yy)
        rank, is_last = plsc.scan_count(bkt)
        cg = plsc.load_gather(cursors, [bkt])
        pos = cg + rank - 1
        plsc.store_scatter(cursors, [bkt], pos + 1, mask=is_last)
        plsc.store_scatter(sx, [pos], xx)
        plsc.store_scatter(sy, [pos], yy)
        plsc.store_scatter(sz, [pos], zz)
        plsc.store_scatter(sidx, [pos], c + j * L + iota)
        return 0

      lax.fori_loop(0, CH // L, sc_body, 0, unroll=2)

    qbase = wid * qw
    pltpu.sync_copy(qb_h.at[pl.ds(qbase, qw)], qbv)
    pltpu.sync_copy(qx_h.at[pl.ds(qbase, qw)], qxv)
    pltpu.sync_copy(qy_h.at[pl.ds(qbase, qw)], qyv)
    pltpu.sync_copy(qz_h.at[pl.ds(qbase, qw)], qzv)

    def cellc(v, hi):
      return jnp.clip((v * (1.0 / CELL)).astype(jnp.int32), 0, hi)

    for t in range(qw // L):
      idxv = t * L + iota
      qbb = plsc.load_gather(qbv, [idxv])
      qxx = plsc.load_gather(qxv, [idxv])
      qyy = plsc.load_gather(qyv, [idxv])
      cxlo = cellc(jnp.maximum(qxx - rv, 0.0), NX - 1)
      cxhi = cellc(jnp.maximum(qxx + rv, 0.0), NX - 1)
      cylo = cellc(jnp.maximum(qyy - rv, 0.0), NY - 1)
      cyhi = cellc(jnp.maximum(qyy + rv, 0.0), NY - 1)
      qbi = qbb.astype(jnp.int32)
      plsc.store_scatter(cxlo_a, [idxv], cxlo)
      plsc.store_scatter(cxhi_a, [idxv], cxhi)
      plsc.store_scatter(base_a, [idxv], qbi * (NX * NY) + cylo)
      plsc.store_scatter(dy1_a, [idxv], cyhi - cylo + 1)

    inf16 = jnp.full((L,), jnp.inf, jnp.float32)
    neg16 = jnp.full((L,), -1, jnp.int32)
    nnvec = nnv[...]

    def q_body(q, _):
      cxlo = cxlo_a[pl.ds(q, L)][0]
      cxhi = cxhi_a[pl.ds(q, L)][0]
      base0 = base_a[pl.ds(q, L)][0]
      dy1 = dy1_a[pl.ds(q, L)][0]
      qf = jnp.full((L,), q, jnp.int32)
      qxb = plsc.load_gather(qxv, [qf])
      qyb = plsc.load_gather(qyv, [qf])
      qzb = plsc.load_gather(qzv, [qf])
      cur_d[...] = inf16
      cur_i[...] = neg16

      def cx_body(cxx, _):
        b0 = base0 + cxx * NY
        s = starts[pl.ds(b0, L)][0]
        e = starts[pl.ds(b0 + dy1, L)][0]

        def w_body(base):
          idxv = base + iota
          m = idxv < e
          idxc = jnp.where(m, idxv, 0)
          xx = plsc.load_gather(sx, [idxc])
          yy = plsc.load_gather(sy, [idxc])
          zz = plsc.load_gather(sz, [idxc])
          dx = xx - qxb
          dy = yy - qyb
          dz = zz - qzb
          d2 = dx * dx + dy * dy + dz * dz
          valid = m & (d2 <= r2v)

          @pl.when(jnp.any(valid))
          def _():
            di = plsc.load_gather(sidx, [idxc])
            cand_d = jnp.where(valid, d2, inf16)
            cand_i = jnp.where(valid, di, neg16)
            cs, civ = plsc.sort_key_val(cand_d, cand_i)
            rd = lax.rev(cur_d[...], (0,))
            ri = lax.rev(cur_i[...], (0,))
            take = cs < rd
            nd, ni = plsc.sort_key_val(
                jnp.minimum(cs, rd), jnp.where(take, civ, ri))
            cur_d[...] = nd
            cur_i[...] = ni

          return base + L

        lax.while_loop(lambda b: b < e, w_body, s)
        return 0

      lax.fori_loop(cxlo, cxhi + 1, cx_body, 0)

      km = (cur_d[...] < jnp.inf) & (iota < nnvec)
      plsc.store_scatter(stage_ri, [q * K + iota],
                         jnp.where(km, cur_i[...], neg16))
      plsc.store_scatter(stage_qi, [q * K + iota],
                         jnp.where(km, qbase + qf, neg16))
      return 0

    with scope("p2_scan"):
      lax.fori_loop(0, qw, q_body, 0)

    pltpu.sync_copy(stage_ri, out_ri.at[pl.ds(qbase * K, qw * K)])
    pltpu.sync_copy(stage_qi, out_qi.at[pl.ds(qbase * K, qw * K)])

  return body


def _build(n_ref, n_query):
  qw = n_query // NW
  mesh = plsc.VectorSubcoreMesh(
      core_axis_name="c", subcore_axis_name="s",
      num_cores=NC, num_subcores=NS)
  scratch = [
      pltpu.VMEM((CH,), jnp.float32),
      pltpu.VMEM((CH,), jnp.float32),
      pltpu.VMEM((CH,), jnp.float32),
      pltpu.VMEM((CH,), jnp.float32),
      pltpu.VMEM((n_ref,), jnp.float32),
      pltpu.VMEM((n_ref,), jnp.float32),
      pltpu.VMEM((n_ref,), jnp.float32),
      pltpu.VMEM((n_ref,), jnp.int32),
      pltpu.VMEM((NBP,), jnp.int32),
      pltpu.VMEM((NBP,), jnp.int32),
      pltpu.VMEM((NBP,), jnp.int32),
      pltpu.VMEM((qw,), jnp.float32),
      pltpu.VMEM((qw,), jnp.float32),
      pltpu.VMEM((qw,), jnp.float32),
      pltpu.VMEM((qw,), jnp.float32),
      pltpu.VMEM((qw + L,), jnp.int32),
      pltpu.VMEM((qw + L,), jnp.int32),
      pltpu.VMEM((qw + L,), jnp.int32),
      pltpu.VMEM((qw + L,), jnp.int32),
      pltpu.VMEM((qw * K,), jnp.int32),
      pltpu.VMEM((qw * K,), jnp.int32),
      pltpu.VMEM((L,), jnp.float32),
      pltpu.VMEM((L,), jnp.int32),
      pltpu.VMEM((L,), jnp.int32),
      pltpu.VMEM((L,), jnp.int32),
      pltpu.VMEM((L,), jnp.float32),
      pltpu.VMEM((L,), jnp.int32),
  ]
  out_type = [
      jax.ShapeDtypeStruct((n_query * K,), jnp.int32),
      jax.ShapeDtypeStruct((n_query * K,), jnp.int32),
  ]
  return pl.kernel(
      _make_body(n_ref, n_query),
      out_type=out_type,
      mesh=mesh,
      scratch_types=scratch,
      compiler_params=pltpu.CompilerParams(needs_layout_passes=False),
  )


def kernel(ref, query, radius, num_neighbors):
  n_ref = ref.shape[0]
  n_query = query.shape[0]
  rb = ref[:, 0]
  rx = ref[:, 1]
  ry = ref[:, 2]
  rz = ref[:, 3]
  qb = query[:, 0]
  qx = query[:, 1]
  qy = query[:, 2]
  qz = query[:, 3]
  rad = jnp.full((L,), radius, jnp.float32)
  nn = jnp.full((L,), num_neighbors, jnp.int32)
  run = _build(n_ref, n_query)
  out_ri, out_qi = run(rb, rx, ry, rz, qb, qx, qy, qz, rad, nn)
  edges = jnp.stack([out_ri, out_qi], axis=0).astype(jnp.int64)
  return edges


# speedup vs baseline: 1.2024x; 1.2024x over previous
import functools

import jax
import jax.numpy as jnp
from jax import lax
from jax.experimental import pallas as pl
from jax.experimental.pallas import tpu as pltpu
from jax.experimental.pallas import tpu_sc as plsc

L = 16
NC, NS = 2, 16
NW = NC * NS
K = 16
CELL = 1.0
NX = 20
NY = 20
NBATCH = 4
NB = NBATCH * NX * NY
NBP = 1664


def _make_body(n_ref, n_query):
  qw = n_query // NW
  nvec = n_ref // L

  def body(rb_h, rx_h, ry_h, rz_h, qb_h, qx_h, qy_h, qz_h, rad_h, nn_h,
           out_ri, out_qi,
           rbf, rxf, ryf, rzf, sidx, barr,
           counts, starts, cursors,
           qbv, qxv, qyv, qzv, cxlo_a, cxhi_a, base_a, dy1_a,
           stage_ri, stage_qi, cur_d, cur_i, tmpa, parv, nnv):
    wid = lax.axis_index("c") * NS + lax.axis_index("s")
    iota = lax.iota(jnp.int32, L)
    zeros16 = jnp.zeros((L,), jnp.int32)

    pltpu.sync_copy(rb_h, rbf)
    pltpu.sync_copy(rx_h, rxf)
    pltpu.sync_copy(ry_h, ryf)
    pltpu.sync_copy(rz_h, rzf)
    pltpu.sync_copy(rad_h, parv)
    pltpu.sync_copy(nn_h, nnv)
    rv = parv[...]
    r2v = rv * rv

    def zero_body(k, _):
      plsc.store_scatter(counts, [k * L + iota], zeros16)
      return 0

    lax.fori_loop(0, NBP // L, zero_body, 0)

    def cnt_body(j, _):
      idxv = j * L + iota
      bb = plsc.load_gather(rbf, [idxv])
      xx = plsc.load_gather(rxf, [idxv])
      yy = plsc.load_gather(ryf, [idxv])
      cx = jnp.clip((xx * (1.0 / CELL)).astype(jnp.int32), 0, NX - 1)
      cy = jnp.clip((yy * (1.0 / CELL)).astype(jnp.int32), 0, NY - 1)
      bkt = (bb.astype(jnp.int32) * NX + cx) * NY + cy
      rank, is_last = plsc.scan_count(bkt)
      plsc.addupdate_scatter(counts, [bkt], rank, mask=is_last)
      packed = bkt | (rank << 16) | jnp.where(is_last, 1 << 21, 0)
      plsc.store_scatter(barr, [idxv], packed)
      return 0

    lax.fori_loop(0, nvec, cnt_body, 0, unroll=4)

    def psum_body(k, carry):
      idxv = k * L + iota
      cnt = plsc.load_gather(counts, [idxv])
      cm = plsc.cumsum(cnt)
      plsc.store_scatter(starts, [idxv], carry + cm - cnt)
      plsc.store_scatter(cursors, [idxv], carry + cm - cnt)
      tmpa[...] = cm
      return carry + plsc.load_gather(tmpa, [jnp.full((L,), L - 1, jnp.int32)])

    lax.fori_loop(0, NBP // L, psum_body, zeros16)

    def pos_body(j, _):
      idxv = j * L + iota
      pk = plsc.load_gather(barr, [idxv])
      bkt = pk & 0xFFFF
      rank = (pk >> 16) & 31
      is_last = (pk >> 21) == 1
      cg = plsc.load_gather(cursors, [bkt])
      pos = cg + rank - 1
      plsc.store_scatter(cursors, [bkt], pos + 1, mask=is_last)
      plsc.store_scatter(sidx, [pos], idxv)
      return 0

    lax.fori_loop(0, nvec, pos_body, 0, unroll=4)

    qbase = wid * qw
    pltpu.sync_copy(qb_h.at[pl.ds(qbase, qw)], qbv)
    pltpu.sync_copy(qx_h.at[pl.ds(qbase, qw)], qxv)
    pltpu.sync_copy(qy_h.at[pl.ds(qbase, qw)], qyv)
    pltpu.sync_copy(qz_h.at[pl.ds(qbase, qw)], qzv)

    def cellc(v, hi):
      return jnp.clip((v * (1.0 / CELL)).astype(jnp.int32), 0, hi)

    for t in range(qw // L):
      idxv = t * L + iota
      qbb = plsc.load_gather(qbv, [idxv])
      qxx = plsc.load_gather(qxv, [idxv])
      qyy = plsc.load_gather(qyv, [idxv])
      cxlo = cellc(jnp.maximum(qxx - rv, 0.0), NX - 1)
      cxhi = cellc(jnp.maximum(qxx + rv, 0.0), NX - 1)
      cylo = cellc(jnp.maximum(qyy - rv, 0.0), NY - 1)
      cyhi = cellc(jnp.maximum(qyy + rv, 0.0), NY - 1)
      qbi = qbb.astype(jnp.int32)
      plsc.store_scatter(cxlo_a, [idxv], cxlo)
      plsc.store_scatter(cxhi_a, [idxv], cxhi)
      plsc.store_scatter(base_a, [idxv], qbi * (NX * NY) + cylo)
      plsc.store_scatter(dy1_a, [idxv], cyhi - cylo + 1)

    inf16 = jnp.full((L,), jnp.inf, jnp.float32)
    neg16 = jnp.full((L,), -1, jnp.int32)
    nnvec = nnv[...]

    def q_body(q, _):
      cxlo = cxlo_a[pl.ds(q, L)][0]
      cxhi = cxhi_a[pl.ds(q, L)][0]
      base0 = base_a[pl.ds(q, L)][0]
      dy1 = dy1_a[pl.ds(q, L)][0]
      qf = jnp.full((L,), q, jnp.int32)
      qxb = plsc.load_gather(qxv, [qf])
      qyb = plsc.load_gather(qyv, [qf])
      qzb = plsc.load_gather(qzv, [qf])
      cur_d[...] = inf16
      cur_i[...] = neg16

      def cx_body(cxx, _):
        b0 = base0 + cxx * NY
        s = starts[pl.ds(b0, L)][0]
        e = starts[pl.ds(b0 + dy1, L)][0]

        def w_body(base):
          idxv = base + iota
          m = idxv < e
          idxc = jnp.where(m, idxv, 0)
          si = plsc.load_gather(sidx, [idxc])
          xx = plsc.load_gather(rxf, [si])
          yy = plsc.load_gather(ryf, [si])
          zz = plsc.load_gather(rzf, [si])
          dx = xx - qxb
          dy = yy - qyb
          dz = zz - qzb
          d2 = dx * dx + dy * dy + dz * dz
          valid = m & (d2 <= r2v)

          @pl.when(jnp.any(valid))
          def _():
            cand_d = jnp.where(valid, d2, inf16)
            cand_i = jnp.where(valid, si, neg16)
            cs, civ = plsc.sort_key_val(cand_d, cand_i)
            rd = lax.rev(cur_d[...], (0,))
            ri = lax.rev(cur_i[...], (0,))
            take = cs < rd
            nd, ni = plsc.sort_key_val(
                jnp.minimum(cs, rd), jnp.where(take, civ, ri))
            cur_d[...] = nd
            cur_i[...] = ni

          return base + L

        lax.while_loop(lambda b: b < e, w_body, s)
        return 0

      lax.fori_loop(cxlo, cxhi + 1, cx_body, 0)

      km = (cur_d[...] < jnp.inf) & (iota < nnvec)
      plsc.store_scatter(stage_ri, [q * K + iota],
                         jnp.where(km, cur_i[...], neg16))
      plsc.store_scatter(stage_qi, [q * K + iota],
                         jnp.where(km, qbase + qf, neg16))
      return 0

    lax.fori_loop(0, qw, q_body, 0)

    pltpu.sync_copy(stage_ri, out_ri.at[pl.ds(qbase * K, qw * K)])
    pltpu.sync_copy(stage_qi, out_qi.at[pl.ds(qbase * K, qw * K)])

  return body


def _build(n_ref, n_query):
  qw = n_query // NW
  mesh = plsc.VectorSubcoreMesh(
      core_axis_name="c", subcore_axis_name="s",
      num_cores=NC, num_subcores=NS)
  scratch = [
      pltpu.VMEM((n_ref,), jnp.float32),
      pltpu.VMEM((n_ref,), jnp.float32),
      pltpu.VMEM((n_ref,), jnp.float32),
      pltpu.VMEM((n_ref,), jnp.float32),
      pltpu.VMEM((n_ref,), jnp.int32),
      pltpu.VMEM((n_ref,), jnp.int32),
      pltpu.VMEM((NBP,), jnp.int32),
      pltpu.VMEM((NBP,), jnp.int32),
      pltpu.VMEM((NBP,), jnp.int32),
      pltpu.VMEM((qw,), jnp.float32),
      pltpu.VMEM((qw,), jnp.float32),
      pltpu.VMEM((qw,), jnp.float32),
      pltpu.VMEM((qw,), jnp.float32),
      pltpu.VMEM((qw + L,), jnp.int32),
      pltpu.VMEM((qw + L,), jnp.int32),
      pltpu.VMEM((qw + L,), jnp.int32),
      pltpu.VMEM((qw + L,), jnp.int32),
      pltpu.VMEM((qw * K,), jnp.int32),
      pltpu.VMEM((qw * K,), jnp.int32),
      pltpu.VMEM((L,), jnp.float32),
      pltpu.VMEM((L,), jnp.int32),
      pltpu.VMEM((L,), jnp.int32),
      pltpu.VMEM((L,), jnp.float32),
      pltpu.VMEM((L,), jnp.int32),
  ]
  out_type = [
      jax.ShapeDtypeStruct((n_query * K,), jnp.int32),
      jax.ShapeDtypeStruct((n_query * K,), jnp.int32),
  ]
  return pl.kernel(
      _make_body(n_ref, n_query),
      out_type=out_type,
      mesh=mesh,
      scratch_types=scratch,
      compiler_params=pltpu.CompilerParams(needs_layout_passes=False),
  )


def kernel(ref, query, radius, num_neighbors):
  n_ref = ref.shape[0]
  n_query = query.shape[0]
  rb = ref[:, 0]
  rx = ref[:, 1]
  ry = ref[:, 2]
  rz = ref[:, 3]
  qb = query[:, 0]
  qx = query[:, 1]
  qy = query[:, 2]
  qz = query[:, 3]
  rad = jnp.full((L,), radius, jnp.float32)
  nn = jnp.full((L,), num_neighbors, jnp.int32)
  run = _build(n_ref, n_query)
  out_ri, out_qi = run(rb, rx, ry, rz, qb, qx, qy, qz, rad, nn)
  edges = jnp.stack([out_ri, out_qi], axis=0).astype(jnp.int64)
  return edges


# speedup vs baseline: 1.3874x; 1.1538x over previous
import functools

import jax
import jax.numpy as jnp
from jax import lax
from jax.experimental import pallas as pl
from jax.experimental.pallas import tpu as pltpu
from jax.experimental.pallas import tpu_sc as plsc

L = 16
NC, NS = 2, 16
NW = NC * NS
K = 16
CELL = 1.0
NX = 20
NY = 20
NBATCH = 4
NB = NBATCH * NX * NY
NBP = 1664


def _make_body(n_ref, n_query):
  qw = n_query // NW
  nvec = n_ref // L

  def body(rb_h, rx_h, ry_h, rz_h, qb_h, qx_h, qy_h, qz_h, rad_h, nn_h,
           out_ri, out_qi,
           rbf, rxf, ryf, rzf, sidx, barr,
           counts, starts, cursors,
           qbv, qxv, qyv, qzv, cxlo_a, cxhi_a, base_a, dy1_a,
           stage_ri, stage_qi, cur_d, cur_i, tmpa, parv, nnv):
    wid = lax.axis_index("c") * NS + lax.axis_index("s")
    iota = lax.iota(jnp.int32, L)
    zeros16 = jnp.zeros((L,), jnp.int32)

    pltpu.sync_copy(rb_h, rbf)
    pltpu.sync_copy(rx_h, rxf)
    pltpu.sync_copy(ry_h, ryf)
    pltpu.sync_copy(rz_h, rzf)
    pltpu.sync_copy(rad_h, parv)
    pltpu.sync_copy(nn_h, nnv)
    rv = parv[...]
    r2v = rv * rv

    def zero_body(k, _):
      plsc.store_scatter(counts, [k * L + iota], zeros16)
      return 0

    lax.fori_loop(0, NBP // L, zero_body, 0)

    @plsc.parallel_loop(0, n_ref, step=L, unroll=4)
    def _(i):
      idxv = i + iota
      bb = plsc.load_gather(rbf, [idxv])
      xx = plsc.load_gather(rxf, [idxv])
      yy = plsc.load_gather(ryf, [idxv])
      cx = jnp.clip((xx * (1.0 / CELL)).astype(jnp.int32), 0, NX - 1)
      cy = jnp.clip((yy * (1.0 / CELL)).astype(jnp.int32), 0, NY - 1)
      bkt = (bb.astype(jnp.int32) * NX + cx) * NY + cy
      rank, is_last = plsc.scan_count(bkt)
      plsc.addupdate_scatter(counts, [bkt], rank, mask=is_last)
      packed = bkt | (rank << 16) | jnp.where(is_last, 1 << 21, 0)
      plsc.store_scatter(barr, [idxv], packed)

    def psum_body(k, carry):
      idxv = k * L + iota
      cnt = plsc.load_gather(counts, [idxv])
      cm = plsc.cumsum(cnt)
      plsc.store_scatter(starts, [idxv], carry + cm - cnt)
      plsc.store_scatter(cursors, [idxv], carry + cm - cnt)
      tmpa[...] = cm
      return carry + plsc.load_gather(tmpa, [jnp.full((L,), L - 1, jnp.int32)])

    lax.fori_loop(0, NBP // L, psum_body, zeros16)

    def pos_body(j, _):
      idxv = j * L + iota
      pk = plsc.load_gather(barr, [idxv])
      bkt = pk & 0xFFFF
      rank = (pk >> 16) & 31
      is_last = (pk >> 21) == 1
      cg = plsc.load_gather(cursors, [bkt])
      pos = cg + rank - 1
      plsc.store_scatter(cursors, [bkt], pos + 1, mask=is_last)
      plsc.store_scatter(sidx, [pos], idxv)
      return 0

    lax.fori_loop(0, nvec, pos_body, 0, unroll=4)

    qbase = wid * qw
    pltpu.sync_copy(qb_h.at[pl.ds(qbase, qw)], qbv)
    pltpu.sync_copy(qx_h.at[pl.ds(qbase, qw)], qxv)
    pltpu.sync_copy(qy_h.at[pl.ds(qbase, qw)], qyv)
    pltpu.sync_copy(qz_h.at[pl.ds(qbase, qw)], qzv)

    def cellc(v, hi):
      return jnp.clip((v * (1.0 / CELL)).astype(jnp.int32), 0, hi)

    for t in range(qw // L):
      idxv = t * L + iota
      qbb = plsc.load_gather(qbv, [idxv])
      qxx = plsc.load_gather(qxv, [idxv])
      qyy = plsc.load_gather(qyv, [idxv])
      cxlo = cellc(jnp.maximum(qxx - rv, 0.0), NX - 1)
      cxhi = cellc(jnp.maximum(qxx + rv, 0.0), NX - 1)
      cylo = cellc(jnp.maximum(qyy - rv, 0.0), NY - 1)
      cyhi = cellc(jnp.maximum(qyy + rv, 0.0), NY - 1)
      qbi = qbb.astype(jnp.int32)
      plsc.store_scatter(cxlo_a, [idxv], cxlo)
      plsc.store_scatter(cxhi_a, [idxv], cxhi)
      plsc.store_scatter(base_a, [idxv], qbi * (NX * NY) + cylo)
      plsc.store_scatter(dy1_a, [idxv], cyhi - cylo + 1)

    inf16 = jnp.full((L,), jnp.inf, jnp.float32)
    neg16 = jnp.full((L,), -1, jnp.int32)
    nnvec = nnv[...]

    def q_body(q, _):
      cxlo = cxlo_a[pl.ds(q, L)][0]
      cxhi = cxhi_a[pl.ds(q, L)][0]
      base0 = base_a[pl.ds(q, L)][0]
      dy1 = dy1_a[pl.ds(q, L)][0]
      qf = jnp.full((L,), q, jnp.int32)
      qxb = plsc.load_gather(qxv, [qf])
      qyb = plsc.load_gather(qyv, [qf])
      qzb = plsc.load_gather(qzv, [qf])
      cur_d[...] = inf16
      cur_i[...] = neg16

      def cx_body(cxx, _):
        b0 = base0 + cxx * NY
        s = starts[pl.ds(b0, L)][0]
        e = starts[pl.ds(b0 + dy1, L)][0]

        def w_body(base):
          idxv = base + iota
          m = idxv < e
          idxc = jnp.where(m, idxv, 0)
          si = plsc.load_gather(sidx, [idxc])
          xx = plsc.load_gather(rxf, [si])
          yy = plsc.load_gather(ryf, [si])
          zz = plsc.load_gather(rzf, [si])
          dx = xx - qxb
          dy = yy - qyb
          dz = zz - qzb
          d2 = dx * dx + dy * dy + dz * dz
          valid = m & (d2 <= r2v)

          @pl.when(jnp.any(valid))
          def _():
            cand_d = jnp.where(valid, d2, inf16)
            cand_i = jnp.where(valid, si, neg16)
            cs, civ = plsc.sort_key_val(cand_d, cand_i)
            rd = lax.rev(cur_d[...], (0,))
            ri = lax.rev(cur_i[...], (0,))
            take = cs < rd
            nd, ni = plsc.sort_key_val(
                jnp.minimum(cs, rd), jnp.where(take, civ, ri))
            cur_d[...] = nd
            cur_i[...] = ni

          return base + L

        lax.while_loop(lambda b: b < e, w_body, s)
        return 0

      lax.fori_loop(cxlo, cxhi + 1, cx_body, 0)

      km = (cur_d[...] < jnp.inf) & (iota < nnvec)
      plsc.store_scatter(stage_ri, [q * K + iota],
                         jnp.where(km, cur_i[...], neg16))
      plsc.store_scatter(stage_qi, [q * K + iota],
                         jnp.where(km, qbase + qf, neg16))
      return 0

    lax.fori_loop(0, qw, q_body, 0)

    pltpu.sync_copy(stage_ri, out_ri.at[pl.ds(qbase * K, qw * K)])
    pltpu.sync_copy(stage_qi, out_qi.at[pl.ds(qbase * K, qw * K)])

  return body


def _build(n_ref, n_query):
  qw = n_query // NW
  mesh = plsc.VectorSubcoreMesh(
      core_axis_name="c", subcore_axis_name="s",
      num_cores=NC, num_subcores=NS)
  scratch = [
      pltpu.VMEM((n_ref,), jnp.float32),
      pltpu.VMEM((n_ref,), jnp.float32),
      pltpu.VMEM((n_ref,), jnp.float32),
      pltpu.VMEM((n_ref,), jnp.float32),
      pltpu.VMEM((n_ref,), jnp.int32),
      pltpu.VMEM((n_ref,), jnp.int32),
      pltpu.VMEM((NBP,), jnp.int32),
      pltpu.VMEM((NBP,), jnp.int32),
      pltpu.VMEM((NBP,), jnp.int32),
      pltpu.VMEM((qw,), jnp.float32),
      pltpu.VMEM((qw,), jnp.float32),
      pltpu.VMEM((qw,), jnp.float32),
      pltpu.VMEM((qw,), jnp.float32),
      pltpu.VMEM((qw + L,), jnp.int32),
      pltpu.VMEM((qw + L,), jnp.int32),
      pltpu.VMEM((qw + L,), jnp.int32),
      pltpu.VMEM((qw + L,), jnp.int32),
      pltpu.VMEM((qw * K,), jnp.int32),
      pltpu.VMEM((qw * K,), jnp.int32),
      pltpu.VMEM((L,), jnp.float32),
      pltpu.VMEM((L,), jnp.int32),
      pltpu.VMEM((L,), jnp.int32),
      pltpu.VMEM((L,), jnp.float32),
      pltpu.VMEM((L,), jnp.int32),
  ]
  out_type = [
      jax.ShapeDtypeStruct((n_query * K,), jnp.int32),
      jax.ShapeDtypeStruct((n_query * K,), jnp.int32),
  ]
  return pl.kernel(
      _make_body(n_ref, n_query),
      out_type=out_type,
      mesh=mesh,
      scratch_types=scratch,
      compiler_params=pltpu.CompilerParams(needs_layout_passes=False),
  )


def kernel(ref, query, radius, num_neighbors):
  n_ref = ref.shape[0]
  n_query = query.shape[0]
  rb = ref[:, 0]
  rx = ref[:, 1]
  ry = ref[:, 2]
  rz = ref[:, 3]
  qb = query[:, 0]
  qx = query[:, 1]
  qy = query[:, 2]
  qz = query[:, 3]
  rad = jnp.full((L,), radius, jnp.float32)
  nn = jnp.full((L,), num_neighbors, jnp.int32)
  run = _build(n_ref, n_query)
  out_ri, out_qi = run(rb, rx, ry, rz, qb, qx, qy, qz, rad, nn)
  edges = jnp.stack([out_ri, out_qi], axis=0).astype(jnp.int64)
  return edges


# speedup vs baseline: 1.3948x; 1.0053x over previous
import functools

import jax
import jax.numpy as jnp
from jax import lax
from jax.experimental import pallas as pl
from jax.experimental.pallas import tpu as pltpu
from jax.experimental.pallas import tpu_sc as plsc

L = 16
NC, NS = 2, 16
NW = NC * NS
K = 16
CELL = 1.0
NX = 20
NY = 20
NBATCH = 4
NB = NBATCH * NX * NY
NBP = 1664


def _make_body(n_ref, n_query):
  qw = n_query // NW
  nvec = n_ref // L

  def body(rb_h, rx_h, ry_h, rz_h, qb_h, qx_h, qy_h, qz_h, rad_h, nn_h,
           out_ri, out_qi,
           rbf, rxf, ryf, rzf, sidx, barr,
           counts, countsb, starts, cursors, cursorsb,
           qbv, qxv, qyv, qzv, cxlo_a, cxhi_a, base_a, dy1_a,
           se1a, se2a, se3a,
           stage_ri, stage_qi, cur_d, cur_i, tmpa, parv, nnv):
    wid = lax.axis_index("c") * NS + lax.axis_index("s")
    iota = lax.iota(jnp.int32, L)
    zeros16 = jnp.zeros((L,), jnp.int32)

    pltpu.sync_copy(rb_h, rbf)
    pltpu.sync_copy(rx_h, rxf)
    pltpu.sync_copy(ry_h, ryf)
    pltpu.sync_copy(rz_h, rzf)
    pltpu.sync_copy(rad_h, parv)
    pltpu.sync_copy(nn_h, nnv)
    rv = parv[...]
    r2v = rv * rv

    def zero_body(k, _):
      plsc.store_scatter(counts, [k * L + iota], zeros16)
      plsc.store_scatter(countsb, [k * L + iota], zeros16)
      return 0

    lax.fori_loop(0, NBP // L, zero_body, 0)

    half = n_ref // 2

    def make_cnt(cnts):
      def cnt_body(i):
        idxv = i + iota
        bb = plsc.load_gather(rbf, [idxv])
        xx = plsc.load_gather(rxf, [idxv])
        yy = plsc.load_gather(ryf, [idxv])
        cx = jnp.clip((xx * (1.0 / CELL)).astype(jnp.int32), 0, NX - 1)
        cy = jnp.clip((yy * (1.0 / CELL)).astype(jnp.int32), 0, NY - 1)
        bkt = (bb.astype(jnp.int32) * NX + cx) * NY + cy
        rank, is_last = plsc.scan_count(bkt)
        plsc.addupdate_scatter(cnts, [bkt], rank, mask=is_last)
        packed = bkt | (rank << 16) | jnp.where(is_last, 1 << 21, 0)
        plsc.store_scatter(barr, [idxv], packed)
      return cnt_body

    plsc.parallel_loop(0, half, step=L, unroll=4)(make_cnt(counts))
    plsc.parallel_loop(half, n_ref, step=L, unroll=4)(make_cnt(countsb))

    def psum_body(k, carry):
      idxv = k * L + iota
      cnta = plsc.load_gather(counts, [idxv])
      cntb = plsc.load_gather(countsb, [idxv])
      cnt = cnta + cntb
      cm = plsc.cumsum(cnt)
      st = carry + cm - cnt
      plsc.store_scatter(starts, [idxv], st)
      plsc.store_scatter(cursors, [idxv], st)
      plsc.store_scatter(cursorsb, [idxv], st + cnta)
      tmpa[...] = cm
      return carry + plsc.load_gather(tmpa, [jnp.full((L,), L - 1, jnp.int32)])

    lax.fori_loop(0, NBP // L, psum_body, zeros16)

    def pos_one(idxv, curs):
      pk = plsc.load_gather(barr, [idxv])
      bkt = pk & 0xFFFF
      rank = (pk >> 16) & 31
      is_last = (pk >> 21) == 1
      cg = plsc.load_gather(curs, [bkt])
      pos = cg + rank - 1
      plsc.store_scatter(curs, [bkt], pos + 1, mask=is_last)
      plsc.store_scatter(sidx, [pos], idxv)

    def pos_body(j, _):
      pos_one(j * L + iota, cursors)
      pos_one(half + j * L + iota, cursorsb)
      return 0

    lax.fori_loop(0, half // L, pos_body, 0, unroll=2)

    qbase = wid * qw
    pltpu.sync_copy(qb_h.at[pl.ds(qbase, qw)], qbv)
    pltpu.sync_copy(qx_h.at[pl.ds(qbase, qw)], qxv)
    pltpu.sync_copy(qy_h.at[pl.ds(qbase, qw)], qyv)
    pltpu.sync_copy(qz_h.at[pl.ds(qbase, qw)], qzv)

    def cellc(v, hi):
      return jnp.clip((v * (1.0 / CELL)).astype(jnp.int32), 0, hi)

    for t in range(qw // L):
      idxv = t * L + iota
      qbb = plsc.load_gather(qbv, [idxv])
      qxx = plsc.load_gather(qxv, [idxv])
      qyy = plsc.load_gather(qyv, [idxv])
      cxlo = cellc(jnp.maximum(qxx - rv, 0.0), NX - 1)
      cxhi = cellc(jnp.maximum(qxx + rv, 0.0), NX - 1)
      cylo = cellc(jnp.maximum(qyy - rv, 0.0), NY - 1)
      cyhi = cellc(jnp.maximum(qyy + rv, 0.0), NY - 1)
      qbi = qbb.astype(jnp.int32)
      base0 = qbi * (NX * NY) + cylo
      dy1 = cyhi - cylo + 1
      plsc.store_scatter(cxlo_a, [idxv], cxlo)
      plsc.store_scatter(cxhi_a, [idxv], cxhi)
      plsc.store_scatter(base_a, [idxv], base0)
      plsc.store_scatter(dy1_a, [idxv], dy1)
      for k, sea in ((0, se1a), (1, se2a), (2, se3a)):
        live = (cxlo + k) <= cxhi
        b0 = base0 + jnp.minimum(cxlo + k, cxhi) * NY
        sk = plsc.load_gather(starts, [b0])
        ek = plsc.load_gather(starts, [b0 + dy1])
        se = jnp.where(live, sk | (ek << 16), 0)
        plsc.store_scatter(sea, [idxv], se)

    inf16 = jnp.full((L,), jnp.inf, jnp.float32)
    neg16 = jnp.full((L,), -1, jnp.int32)
    nnvec = nnv[...]

    def q_body(q, _):
      qf = jnp.full((L,), q, jnp.int32)
      qxb = plsc.load_gather(qxv, [qf])
      qyb = plsc.load_gather(qyv, [qf])
      qzb = plsc.load_gather(qzv, [qf])
      cur_d[...] = inf16
      cur_i[...] = neg16

      def scan_strip(s, e):
        def w_body(base):
          idxv = base + iota
          m = idxv < e
          idxc = jnp.where(m, idxv, 0)
          si = plsc.load_gather(sidx, [idxc])
          xx = plsc.load_gather(rxf, [si])
          yy = plsc.load_gather(ryf, [si])
          zz = plsc.load_gather(rzf, [si])
          dx = xx - qxb
          dy = yy - qyb
          dz = zz - qzb
          d2 = dx * dx + dy * dy + dz * dz
          valid = m & (d2 <= r2v)

          @pl.when(jnp.any(valid))
          def _():
            cand_d = jnp.where(valid, d2, inf16)
            cand_i = jnp.where(valid, si, neg16)
            cs, civ = plsc.sort_key_val(cand_d, cand_i)
            rd = lax.rev(cur_d[...], (0,))
            ri = lax.rev(cur_i[...], (0,))
            take = cs < rd
            nd, ni = plsc.sort_key_val(
                jnp.minimum(cs, rd), jnp.where(take, civ, ri))
            cur_d[...] = nd
            cur_i[...] = ni

          return base + L

        lax.while_loop(lambda b: b < e, w_body, s)

      for sea in (se1a, se2a, se3a):
        se = sea[pl.ds(q, L)][0]
        scan_strip(se & 0xFFFF, se >> 16)

      cxlo = cxlo_a[pl.ds(q, L)][0]
      cxhi = cxhi_a[pl.ds(q, L)][0]

      @pl.when(cxhi - cxlo > 2)
      def _():
        base0 = base_a[pl.ds(q, L)][0]
        dy1 = dy1_a[pl.ds(q, L)][0]

        def cx_body(cxx, _):
          b0 = base0 + cxx * NY
          scan_strip(starts[pl.ds(b0, L)][0], starts[pl.ds(b0 + dy1, L)][0])
          return 0

        lax.fori_loop(cxlo + 3, cxhi + 1, cx_body, 0)

      km = (cur_d[...] < jnp.inf) & (iota < nnvec)
      plsc.store_scatter(stage_ri, [q * K + iota],
                         jnp.where(km, cur_i[...], neg16))
      plsc.store_scatter(stage_qi, [q * K + iota],
                         jnp.where(km, qbase + qf, neg16))
      return 0

    lax.fori_loop(0, qw, q_body, 0)

    pltpu.sync_copy(stage_ri, out_ri.at[pl.ds(qbase * K, qw * K)])
    pltpu.sync_copy(stage_qi, out_qi.at[pl.ds(qbase * K, qw * K)])

  return body


def _build(n_ref, n_query):
  qw = n_query // NW
  mesh = plsc.VectorSubcoreMesh(
      core_axis_name="c", subcore_axis_name="s",
      num_cores=NC, num_subcores=NS)
  scratch = [
      pltpu.VMEM((n_ref,), jnp.float32),
      pltpu.VMEM((n_ref,), jnp.float32),
      pltpu.VMEM((n_ref,), jnp.float32),
      pltpu.VMEM((n_ref,), jnp.float32),
      pltpu.VMEM((n_ref,), jnp.int32),
      pltpu.VMEM((n_ref,), jnp.int32),
      pltpu.VMEM((NBP,), jnp.int32),
      pltpu.VMEM((NBP,), jnp.int32),
      pltpu.VMEM((NBP,), jnp.int32),
      pltpu.VMEM((NBP,), jnp.int32),
      pltpu.VMEM((NBP,), jnp.int32),
      pltpu.VMEM((qw,), jnp.float32),
      pltpu.VMEM((qw,), jnp.float32),
      pltpu.VMEM((qw,), jnp.float32),
      pltpu.VMEM((qw,), jnp.float32),
      pltpu.VMEM((qw + L,), jnp.int32),
      pltpu.VMEM((qw + L,), jnp.int32),
      pltpu.VMEM((qw + L,), jnp.int32),
      pltpu.VMEM((qw + L,), jnp.int32),
      pltpu.VMEM((qw + L,), jnp.int32),
      pltpu.VMEM((qw + L,), jnp.int32),
      pltpu.VMEM((qw + L,), jnp.int32),
      pltpu.VMEM((qw * K,), jnp.int32),
      pltpu.VMEM((qw * K,), jnp.int32),
      pltpu.VMEM((L,), jnp.float32),
      pltpu.VMEM((L,), jnp.int32),
      pltpu.VMEM((L,), jnp.int32),
      pltpu.VMEM((L,), jnp.float32),
      pltpu.VMEM((L,), jnp.int32),
  ]
  out_type = [
      jax.ShapeDtypeStruct((n_query * K,), jnp.int32),
      jax.ShapeDtypeStruct((n_query * K,), jnp.int32),
  ]
  return pl.kernel(
      _make_body(n_ref, n_query),
      out_type=out_type,
      mesh=mesh,
      scratch_types=scratch,
      compiler_params=pltpu.CompilerParams(needs_layout_passes=False),
  )


def kernel(ref, query, radius, num_neighbors):
  n_ref = ref.shape[0]
  n_query = query.shape[0]
  rb = ref[:, 0]
  rx = ref[:, 1]
  ry = ref[:, 2]
  rz = ref[:, 3]
  qb = query[:, 0]
  qx = query[:, 1]
  qy = query[:, 2]
  qz = query[:, 3]
  rad = jnp.full((L,), radius, jnp.float32)
  nn = jnp.full((L,), num_neighbors, jnp.int32)
  run = _build(n_ref, n_query)
  out_ri, out_qi = run(rb, rx, ry, rz, qb, qx, qy, qz, rad, nn)
  edges = jnp.stack([out_ri, out_qi], axis=0).astype(jnp.int64)
  return edges


# speedup vs baseline: 1.4926x; 1.0701x over previous
import functools

import jax
import jax.numpy as jnp
from jax import lax
from jax.experimental import pallas as pl
from jax.experimental.pallas import tpu as pltpu
from jax.experimental.pallas import tpu_sc as plsc

L = 16
NC, NS = 2, 16
NW = NC * NS
K = 16
CELL = 1.0
NX = 20
NY = 20
NBATCH = 4
NB = NBATCH * NX * NY
NBP = 1664


def _make_body(n_ref, n_query):
  qw = n_query // NW
  nvec = n_ref // L

  def body(rb_h, rx_h, ry_h, rz_h, qb_h, qx_h, qy_h, qz_h, rad_h, nn_h,
           out_ri, out_qi,
           rbf, rxf, ryf, rzf, sidx, barr,
           counts, countsb, starts, cursors, cursorsb,
           qbv, qxv, qyv, qzv, cxlo_a, cxhi_a, base_a, dy1_a,
           se1a, se2a, se3a,
           stage_ri, stage_qi, cur_d, cur_i, tmpa, parv, nnv):
    wid = lax.axis_index("c") * NS + lax.axis_index("s")
    iota = lax.iota(jnp.int32, L)
    zeros16 = jnp.zeros((L,), jnp.int32)

    pltpu.sync_copy(rb_h, rbf)
    pltpu.sync_copy(rx_h, rxf)
    pltpu.sync_copy(ry_h, ryf)
    pltpu.sync_copy(rz_h, rzf)
    pltpu.sync_copy(rad_h, parv)
    pltpu.sync_copy(nn_h, nnv)
    rv = parv[...]
    r2v = rv * rv

    def zero_body(k, _):
      plsc.store_scatter(counts, [k * L + iota], zeros16)
      plsc.store_scatter(countsb, [k * L + iota], zeros16)
      return 0

    lax.fori_loop(0, NBP // L, zero_body, 0)

    half = n_ref // 2

    def make_cnt(cnts):
      def cnt_body(i):
        idxv = i + iota
        bb = plsc.load_gather(rbf, [idxv])
        xx = plsc.load_gather(rxf, [idxv])
        yy = plsc.load_gather(ryf, [idxv])
        cx = jnp.clip((xx * (1.0 / CELL)).astype(jnp.int32), 0, NX - 1)
        cy = jnp.clip((yy * (1.0 / CELL)).astype(jnp.int32), 0, NY - 1)
        bkt = (bb.astype(jnp.int32) * NX + cx) * NY + cy
        rank, is_last = plsc.scan_count(bkt)
        plsc.addupdate_scatter(cnts, [bkt], rank, mask=is_last)
        packed = bkt | (rank << 16) | jnp.where(is_last, 1 << 21, 0)
        plsc.store_scatter(barr, [idxv], packed)
      return cnt_body

    plsc.parallel_loop(0, half, step=L, unroll=4)(make_cnt(counts))
    plsc.parallel_loop(half, n_ref, step=L, unroll=4)(make_cnt(countsb))

    def psum_body(k, carry):
      idxv = k * L + iota
      cnta = plsc.load_gather(counts, [idxv])
      cntb = plsc.load_gather(countsb, [idxv])
      cnt = cnta + cntb
      cm = plsc.cumsum(cnt)
      st = carry + cm - cnt
      plsc.store_scatter(starts, [idxv], st)
      plsc.store_scatter(cursors, [idxv], st)
      plsc.store_scatter(cursorsb, [idxv], st + cnta)
      tmpa[...] = cm
      return carry + plsc.load_gather(tmpa, [jnp.full((L,), L - 1, jnp.int32)])

    lax.fori_loop(0, NBP // L, psum_body, zeros16)

    def pos_one(idxv, curs):
      pk = plsc.load_gather(barr, [idxv])
      bkt = pk & 0xFFFF
      rank = (pk >> 16) & 31
      is_last = (pk >> 21) == 1
      cg = plsc.load_gather(curs, [bkt])
      pos = cg + rank - 1
      plsc.store_scatter(curs, [bkt], pos + 1, mask=is_last)
      plsc.store_scatter(sidx, [pos], idxv)

    def pos_body(j, _):
      pos_one(j * L + iota, cursors)
      pos_one(half + j * L + iota, cursorsb)
      return 0

    lax.fori_loop(0, half // L, pos_body, 0, unroll=2)

    qbase = wid * qw
    pltpu.sync_copy(qb_h.at[pl.ds(qbase, qw)], qbv)
    pltpu.sync_copy(qx_h.at[pl.ds(qbase, qw)], qxv)
    pltpu.sync_copy(qy_h.at[pl.ds(qbase, qw)], qyv)
    pltpu.sync_copy(qz_h.at[pl.ds(qbase, qw)], qzv)

    def cellc(v, hi):
      return jnp.clip((v * (1.0 / CELL)).astype(jnp.int32), 0, hi)

    for t in range(qw // L):
      idxv = t * L + iota
      qbb = plsc.load_gather(qbv, [idxv])
      qxx = plsc.load_gather(qxv, [idxv])
      qyy = plsc.load_gather(qyv, [idxv])
      cxlo = cellc(jnp.maximum(qxx - rv, 0.0), NX - 1)
      cxhi = cellc(jnp.maximum(qxx + rv, 0.0), NX - 1)
      cylo = cellc(jnp.maximum(qyy - rv, 0.0), NY - 1)
      cyhi = cellc(jnp.maximum(qyy + rv, 0.0), NY - 1)
      qbi = qbb.astype(jnp.int32)
      base0 = qbi * (NX * NY) + cylo
      dy1 = cyhi - cylo + 1
      plsc.store_scatter(cxlo_a, [idxv], cxlo)
      plsc.store_scatter(cxhi_a, [idxv], cxhi)
      plsc.store_scatter(base_a, [idxv], base0)
      plsc.store_scatter(dy1_a, [idxv], dy1)
      for k, sea in ((0, se1a), (1, se2a), (2, se3a)):
        live = (cxlo + k) <= cxhi
        b0 = base0 + jnp.minimum(cxlo + k, cxhi) * NY
        sk = plsc.load_gather(starts, [b0])
        ek = plsc.load_gather(starts, [b0 + dy1])
        se = jnp.where(live, sk | (ek << 16), 0)
        plsc.store_scatter(sea, [idxv], se)

    inf16 = jnp.full((L,), jnp.inf, jnp.float32)
    neg16 = jnp.full((L,), -1, jnp.int32)
    nnvec = nnv[...]

    def q_body(q, _):
      qf = jnp.full((L,), q, jnp.int32)
      qxb = plsc.load_gather(qxv, [qf])
      qyb = plsc.load_gather(qyv, [qf])
      qzb = plsc.load_gather(qzv, [qf])
      cur_d[...] = inf16
      cur_i[...] = neg16

      def scan_strip(s, e):
        def probe(idxv):
          m = idxv < e
          idxc = jnp.where(m, idxv, 0)
          si = plsc.load_gather(sidx, [idxc])
          xx = plsc.load_gather(rxf, [si])
          yy = plsc.load_gather(ryf, [si])
          zz = plsc.load_gather(rzf, [si])
          dx = xx - qxb
          dy = yy - qyb
          dz = zz - qzb
          d2 = dx * dx + dy * dy + dz * dz
          return m & (d2 <= r2v), d2, si

        def merge(valid, d2, si):
          @pl.when(jnp.any(valid))
          def _():
            cand_d = jnp.where(valid, d2, inf16)
            cand_i = jnp.where(valid, si, neg16)
            cs, civ = plsc.sort_key_val(cand_d, cand_i)
            rd = lax.rev(cur_d[...], (0,))
            ri = lax.rev(cur_i[...], (0,))
            take = cs < rd
            nd, ni = plsc.sort_key_val(
                jnp.minimum(cs, rd), jnp.where(take, civ, ri))
            cur_d[...] = nd
            cur_i[...] = ni

        def w_body(base):
          v0, d0, s0 = probe(base + iota)
          v1, d1, s1 = probe(base + L + iota)
          merge(v0, d0, s0)
          merge(v1, d1, s1)
          return base + 2 * L

        lax.while_loop(lambda b: b < e, w_body, s)

      for sea in (se1a, se2a, se3a):
        se = sea[pl.ds(q, L)][0]
        scan_strip(se & 0xFFFF, se >> 16)

      cxlo = cxlo_a[pl.ds(q, L)][0]
      cxhi = cxhi_a[pl.ds(q, L)][0]

      @pl.when(cxhi - cxlo > 2)
      def _():
        base0 = base_a[pl.ds(q, L)][0]
        dy1 = dy1_a[pl.ds(q, L)][0]

        def cx_body(cxx, _):
          b0 = base0 + cxx * NY
          scan_strip(starts[pl.ds(b0, L)][0], starts[pl.ds(b0 + dy1, L)][0])
          return 0

        lax.fori_loop(cxlo + 3, cxhi + 1, cx_body, 0)

      km = (cur_d[...] < jnp.inf) & (iota < nnvec)
      plsc.store_scatter(stage_ri, [q * K + iota],
                         jnp.where(km, cur_i[...], neg16))
      plsc.store_scatter(stage_qi, [q * K + iota],
                         jnp.where(km, qbase + qf, neg16))
      return 0

    lax.fori_loop(0, qw, q_body, 0)

    pltpu.sync_copy(stage_ri, out_ri.at[pl.ds(qbase * K, qw * K)])
    pltpu.sync_copy(stage_qi, out_qi.at[pl.ds(qbase * K, qw * K)])

  return body


def _build(n_ref, n_query):
  qw = n_query // NW
  mesh = plsc.VectorSubcoreMesh(
      core_axis_name="c", subcore_axis_name="s",
      num_cores=NC, num_subcores=NS)
  scratch = [
      pltpu.VMEM((n_ref,), jnp.float32),
      pltpu.VMEM((n_ref,), jnp.float32),
      pltpu.VMEM((n_ref,), jnp.float32),
      pltpu.VMEM((n_ref,), jnp.float32),
      pltpu.VMEM((n_ref,), jnp.int32),
      pltpu.VMEM((n_ref,), jnp.int32),
      pltpu.VMEM((NBP,), jnp.int32),
      pltpu.VMEM((NBP,), jnp.int32),
      pltpu.VMEM((NBP,), jnp.int32),
      pltpu.VMEM((NBP,), jnp.int32),
      pltpu.VMEM((NBP,), jnp.int32),
      pltpu.VMEM((qw,), jnp.float32),
      pltpu.VMEM((qw,), jnp.float32),
      pltpu.VMEM((qw,), jnp.float32),
      pltpu.VMEM((qw,), jnp.float32),
      pltpu.VMEM((qw + L,), jnp.int32),
      pltpu.VMEM((qw + L,), jnp.int32),
      pltpu.VMEM((qw + L,), jnp.int32),
      pltpu.VMEM((qw + L,), jnp.int32),
      pltpu.VMEM((qw + L,), jnp.int32),
      pltpu.VMEM((qw + L,), jnp.int32),
      pltpu.VMEM((qw + L,), jnp.int32),
      pltpu.VMEM((qw * K,), jnp.int32),
      pltpu.VMEM((qw * K,), jnp.int32),
      pltpu.VMEM((L,), jnp.float32),
      pltpu.VMEM((L,), jnp.int32),
      pltpu.VMEM((L,), jnp.int32),
      pltpu.VMEM((L,), jnp.float32),
      pltpu.VMEM((L,), jnp.int32),
  ]
  out_type = [
      jax.ShapeDtypeStruct((n_query * K,), jnp.int32),
      jax.ShapeDtypeStruct((n_query * K,), jnp.int32),
  ]
  return pl.kernel(
      _make_body(n_ref, n_query),
      out_type=out_type,
      mesh=mesh,
      scratch_types=scratch,
      compiler_params=pltpu.CompilerParams(needs_layout_passes=False),
  )


def kernel(ref, query, radius, num_neighbors):
  n_ref = ref.shape[0]
  n_query = query.shape[0]
  rb = ref[:, 0]
  rx = ref[:, 1]
  ry = ref[:, 2]
  rz = ref[:, 3]
  qb = query[:, 0]
  qx = query[:, 1]
  qy = query[:, 2]
  qz = query[:, 3]
  rad = jnp.full((L,), radius, jnp.float32)
  nn = jnp.full((L,), num_neighbors, jnp.int32)
  run = _build(n_ref, n_query)
  out_ri, out_qi = run(rb, rx, ry, rz, qb, qx, qy, qz, rad, nn)
  edges = jnp.stack([out_ri, out_qi], axis=0).astype(jnp.int64)
  return edges


# speedup vs baseline: 1.6112x; 1.0795x over previous
import functools

import jax
import jax.numpy as jnp
from jax import lax
from jax.experimental import pallas as pl
from jax.experimental.pallas import tpu as pltpu
from jax.experimental.pallas import tpu_sc as plsc

L = 16
NC, NS = 2, 16
NW = NC * NS
K = 16
CELL = 1.0
NX = 20
NY = 20
NBATCH = 4
NB = NBATCH * NX * NY
NBP = 1664


def _make_body(n_ref, n_query):
  qw = n_query // NW
  nvec = n_ref // L

  def body(rb_h, rx_h, ry_h, rz_h, qb_h, qx_h, qy_h, qz_h, rad_h, nn_h,
           out_ri, out_qi,
           rbf, rxf, ryf, rzf, sidx, barr,
           counts, countsb, starts, cursors, cursorsb,
           qbv, qxv, qyv, qzv, cxlo_a, cxhi_a, base_a, dy1_a,
           se1a, se2a, se3a,
           stage_ri, stage_qi, cur_d, cur_i, tmpa, parv, nnv):
    wid = lax.axis_index("c") * NS + lax.axis_index("s")
    iota = lax.iota(jnp.int32, L)
    zeros16 = jnp.zeros((L,), jnp.int32)

    pltpu.sync_copy(rb_h, rbf)
    pltpu.sync_copy(rx_h, rxf)
    pltpu.sync_copy(ry_h, ryf)
    pltpu.sync_copy(rz_h, rzf)
    pltpu.sync_copy(rad_h, parv)
    pltpu.sync_copy(nn_h, nnv)
    rv = parv[...]
    r2v = rv * rv

    def zero_body(k, _):
      plsc.store_scatter(counts, [k * L + iota], zeros16)
      plsc.store_scatter(countsb, [k * L + iota], zeros16)
      return 0

    lax.fori_loop(0, NBP // L, zero_body, 0)

    half = n_ref // 2

    def make_cnt(cnts):
      def cnt_body(i):
        idxv = i + iota
        bb = plsc.load_gather(rbf, [idxv])
        xx = plsc.load_gather(rxf, [idxv])
        yy = plsc.load_gather(ryf, [idxv])
        cx = jnp.clip((xx * (1.0 / CELL)).astype(jnp.int32), 0, NX - 1)
        cy = jnp.clip((yy * (1.0 / CELL)).astype(jnp.int32), 0, NY - 1)
        bkt = (bb.astype(jnp.int32) * NX + cx) * NY + cy
        rank, is_last = plsc.scan_count(bkt)
        plsc.addupdate_scatter(cnts, [bkt], rank, mask=is_last)
        packed = bkt | (rank << 16) | jnp.where(is_last, 1 << 21, 0)
        plsc.store_scatter(barr, [idxv], packed)
      return cnt_body

    plsc.parallel_loop(0, half, step=L, unroll=4)(make_cnt(counts))
    plsc.parallel_loop(half, n_ref, step=L, unroll=4)(make_cnt(countsb))

    def psum_body(k, carry):
      idxv = k * L + iota
      cnta = plsc.load_gather(counts, [idxv])
      cntb = plsc.load_gather(countsb, [idxv])
      cnt = cnta + cntb
      cm = plsc.cumsum(cnt)
      st = carry + cm - cnt
      plsc.store_scatter(starts, [idxv], st)
      plsc.store_scatter(cursors, [idxv], st)
      plsc.store_scatter(cursorsb, [idxv], st + cnta)
      tmpa[...] = cm
      return carry + plsc.load_gather(tmpa, [jnp.full((L,), L - 1, jnp.int32)])

    lax.fori_loop(0, NBP // L, psum_body, zeros16)

    def pos_one(idxv, curs):
      pk = plsc.load_gather(barr, [idxv])
      bkt = pk & 0xFFFF
      rank = (pk >> 16) & 31
      is_last = (pk >> 21) == 1
      cg = plsc.load_gather(curs, [bkt])
      pos = cg + rank - 1
      plsc.store_scatter(curs, [bkt], pos + 1, mask=is_last)
      plsc.store_scatter(sidx, [pos], idxv)

    def pos_body(j, _):
      pos_one(j * L + iota, cursors)
      pos_one(half + j * L + iota, cursorsb)
      return 0

    lax.fori_loop(0, half // L, pos_body, 0, unroll=2)

    qbase = wid * qw
    pltpu.sync_copy(qb_h.at[pl.ds(qbase, qw)], qbv)
    pltpu.sync_copy(qx_h.at[pl.ds(qbase, qw)], qxv)
    pltpu.sync_copy(qy_h.at[pl.ds(qbase, qw)], qyv)
    pltpu.sync_copy(qz_h.at[pl.ds(qbase, qw)], qzv)

    def cellc(v, hi):
      return jnp.clip((v * (1.0 / CELL)).astype(jnp.int32), 0, hi)

    for t in range(qw // L):
      idxv = t * L + iota
      qbb = plsc.load_gather(qbv, [idxv])
      qxx = plsc.load_gather(qxv, [idxv])
      qyy = plsc.load_gather(qyv, [idxv])
      cxlo = cellc(jnp.maximum(qxx - rv, 0.0), NX - 1)
      cxhi = cellc(jnp.maximum(qxx + rv, 0.0), NX - 1)
      cylo = cellc(jnp.maximum(qyy - rv, 0.0), NY - 1)
      cyhi = cellc(jnp.maximum(qyy + rv, 0.0), NY - 1)
      qbi = qbb.astype(jnp.int32)
      base0 = qbi * (NX * NY) + cylo
      dy1 = cyhi - cylo + 1
      plsc.store_scatter(cxlo_a, [idxv], cxlo)
      plsc.store_scatter(cxhi_a, [idxv], cxhi)
      plsc.store_scatter(base_a, [idxv], base0)
      plsc.store_scatter(dy1_a, [idxv], dy1)
      for k, sea in ((0, se1a), (1, se2a), (2, se3a)):
        live = (cxlo + k) <= cxhi
        b0 = base0 + jnp.minimum(cxlo + k, cxhi) * NY
        sk = plsc.load_gather(starts, [b0])
        ek = plsc.load_gather(starts, [b0 + dy1])
        se = jnp.where(live, sk | (ek << 16), 0)
        plsc.store_scatter(sea, [idxv], se)

    inf16 = jnp.full((L,), jnp.inf, jnp.float32)
    neg16 = jnp.full((L,), -1, jnp.int32)
    nnvec = nnv[...]

    def q_body(q, _):
      qf = jnp.full((L,), q, jnp.int32)
      qxb = plsc.load_gather(qxv, [qf])
      qyb = plsc.load_gather(qyv, [qf])
      qzb = plsc.load_gather(qzv, [qf])
      cur_d[...] = inf16
      cur_i[...] = neg16

      def scan_strip(s, e):
        def probe(idxv):
          m = idxv < e
          idxc = jnp.where(m, idxv, 0)
          si = plsc.load_gather(sidx, [idxc])
          xx = plsc.load_gather(rxf, [si])
          yy = plsc.load_gather(ryf, [si])
          zz = plsc.load_gather(rzf, [si])
          dx = xx - qxb
          dy = yy - qyb
          dz = zz - qzb
          d2 = dx * dx + dy * dy + dz * dz
          return m & (d2 <= r2v), d2, si

        def merge(valid, d2, si):
          cand_d = jnp.where(valid, d2, inf16)
          cs, civ = plsc.sort_key_val(cand_d, si)
          rd = lax.rev(cur_d[...], (0,))
          ri = lax.rev(cur_i[...], (0,))
          take = cs < rd
          nd, ni = plsc.sort_key_val(
              jnp.minimum(cs, rd), jnp.where(take, civ, ri))
          cur_d[...] = nd
          cur_i[...] = ni

        def w_body(base):
          v0, d0, s0 = probe(base + iota)
          v1, d1, s1 = probe(base + L + iota)
          merge(v0, d0, s0)
          merge(v1, d1, s1)
          return base + 2 * L

        lax.while_loop(lambda b: b < e, w_body, s)

      for sea in (se1a, se2a, se3a):
        se = sea[pl.ds(q, L)][0]
        scan_strip(se & 0xFFFF, se >> 16)

      cxlo = cxlo_a[pl.ds(q, L)][0]
      cxhi = cxhi_a[pl.ds(q, L)][0]

      @pl.when(cxhi - cxlo > 2)
      def _():
        base0 = base_a[pl.ds(q, L)][0]
        dy1 = dy1_a[pl.ds(q, L)][0]

        def cx_body(cxx, _):
          b0 = base0 + cxx * NY
          scan_strip(starts[pl.ds(b0, L)][0], starts[pl.ds(b0 + dy1, L)][0])
          return 0

        lax.fori_loop(cxlo + 3, cxhi + 1, cx_body, 0)

      km = (cur_d[...] < jnp.inf) & (iota < nnvec)
      plsc.store_scatter(stage_ri, [q * K + iota],
                         jnp.where(km, cur_i[...], neg16))
      plsc.store_scatter(stage_qi, [q * K + iota],
                         jnp.where(km, qbase + qf, neg16))
      return 0

    lax.fori_loop(0, qw, q_body, 0)

    pltpu.sync_copy(stage_ri, out_ri.at[pl.ds(qbase * K, qw * K)])
    pltpu.sync_copy(stage_qi, out_qi.at[pl.ds(qbase * K, qw * K)])

  return body


def _build(n_ref, n_query):
  qw = n_query // NW
  mesh = plsc.VectorSubcoreMesh(
      core_axis_name="c", subcore_axis_name="s",
      num_cores=NC, num_subcores=NS)
  scratch = [
      pltpu.VMEM((n_ref,), jnp.float32),
      pltpu.VMEM((n_ref,), jnp.float32),
      pltpu.VMEM((n_ref,), jnp.float32),
      pltpu.VMEM((n_ref,), jnp.float32),
      pltpu.VMEM((n_ref,), jnp.int32),
      pltpu.VMEM((n_ref,), jnp.int32),
      pltpu.VMEM((NBP,), jnp.int32),
      pltpu.VMEM((NBP,), jnp.int32),
      pltpu.VMEM((NBP,), jnp.int32),
      pltpu.VMEM((NBP,), jnp.int32),
      pltpu.VMEM((NBP,), jnp.int32),
      pltpu.VMEM((qw,), jnp.float32),
      pltpu.VMEM((qw,), jnp.float32),
      pltpu.VMEM((qw,), jnp.float32),
      pltpu.VMEM((qw,), jnp.float32),
      pltpu.VMEM((qw + L,), jnp.int32),
      pltpu.VMEM((qw + L,), jnp.int32),
      pltpu.VMEM((qw + L,), jnp.int32),
      pltpu.VMEM((qw + L,), jnp.int32),
      pltpu.VMEM((qw + L,), jnp.int32),
      pltpu.VMEM((qw + L,), jnp.int32),
      pltpu.VMEM((qw + L,), jnp.int32),
      pltpu.VMEM((qw * K,), jnp.int32),
      pltpu.VMEM((qw * K,), jnp.int32),
      pltpu.VMEM((L,), jnp.float32),
      pltpu.VMEM((L,), jnp.int32),
      pltpu.VMEM((L,), jnp.int32),
      pltpu.VMEM((L,), jnp.float32),
      pltpu.VMEM((L,), jnp.int32),
  ]
  out_type = [
      jax.ShapeDtypeStruct((n_query * K,), jnp.int32),
      jax.ShapeDtypeStruct((n_query * K,), jnp.int32),
  ]
  return pl.kernel(
      _make_body(n_ref, n_query),
      out_type=out_type,
      mesh=mesh,
      scratch_types=scratch,
      compiler_params=pltpu.CompilerParams(needs_layout_passes=False),
  )


def kernel(ref, query, radius, num_neighbors):
  n_ref = ref.shape[0]
  n_query = query.shape[0]
  rb = ref[:, 0]
  rx = ref[:, 1]
  ry = ref[:, 2]
  rz = ref[:, 3]
  qb = query[:, 0]
  qx = query[:, 1]
  qy = query[:, 2]
  qz = query[:, 3]
  rad = jnp.full((L,), radius, jnp.float32)
  nn = jnp.full((L,), num_neighbors, jnp.int32)
  run = _build(n_ref, n_query)
  out_ri, out_qi = run(rb, rx, ry, rz, qb, qx, qy, qz, rad, nn)
  edges = jnp.stack([out_ri, out_qi], axis=0).astype(jnp.int64)
  return edges
